# baseline (speedup 1.0000x reference)
import jax
import jax.numpy as jnp
from jax import lax
from jax.experimental import pallas as pl
from jax.experimental.pallas import tpu as pltpu
from jax.experimental.pallas import tpu_sc as plsc

B = 16384
D = 128
NSP = 26
DENSE = 256
NC = 2
NS = 16
NW = NC * NS
RPW = B // NW
CH = 64
NCH = RPW // CH
TPW = NSP * NCH
NB = 4
NPLAIN = 3 * NCH


def _sc_emb_body(uids, iids, spt, utab, itab, ttab,
                 u_out, i_out, t_out,
                 idxu, idxi, tidx, gb0, gb1, gb2, gb3, acc,
                 tsem, isem0, isem1,
                 gsem0, gsem1, gsem2, gsem3,
                 ssem0, ssem1, ssem2, ssem3,
                 asem0, asem1, asem2, asem3,
                 asem4, asem5, asem6, asem7):
  c = lax.axis_index("c")
  s = lax.axis_index("s")
  wid = s * NC + c
  base = wid * RPW
  bufs = (gb0, gb1, gb2, gb3)
  gsems = (gsem0, gsem1, gsem2, gsem3)
  ssems = (ssem0, ssem1, ssem2, ssem3)
  asems = (asem0, asem1, asem2, asem3, asem4, asem5, asem6, asem7)

  pltpu.async_copy(spt.at[pl.ds(wid * TPW, TPW)], tidx, tsem)
  pltpu.async_copy(uids.at[pl.ds(wid * NCH, NCH)], idxu, isem0)
  pltpu.async_copy(iids.at[pl.ds(wid * NCH, NCH)], idxi, isem1)
  pltpu.make_async_copy(uids.at[pl.ds(wid * NCH, NCH)], idxu, isem0).wait()
  pltpu.make_async_copy(iids.at[pl.ds(wid * NCH, NCH)], idxi, isem1).wait()
  pltpu.make_async_copy(spt.at[pl.ds(wid * TPW, TPW)], tidx, tsem).wait()

  def a_dst(b):
    return acc.at[pl.ds(b * CH, CH), :]

  for b in range(NCH):
    pltpu.async_copy(ttab.at[tidx.at[b]], a_dst(b), asems[b])

  def g_src(k):
    if k < NCH:
      return utab.at[idxu.at[k]]
    return itab.at[idxi.at[k - NCH]]

  def o_dst(k):
    if k < NCH:
      return u_out.at[pl.ds(base + k * CH, CH), :]
    return i_out.at[pl.ds(base + (k - NCH) * CH, CH), :]

  NUI = 2 * NCH
  for b in range(NB):
    pltpu.async_copy(g_src(b), bufs[b], gsems[b])
  for k0 in range(0, NUI, NB):
    for b in range(NB):
      k = k0 + b
      pltpu.make_async_copy(g_src(k), bufs[b], gsems[b]).wait()
      pltpu.async_copy(bufs[b], o_dst(k), ssems[b])
    for b in range(NB):
      k = k0 + b
      pltpu.make_async_copy(bufs[b], o_dst(k), ssems[b]).wait()
      if k + NB < NUI:
        pltpu.async_copy(g_src(k + NB), bufs[b], gsems[b])

  for b in range(NCH):
    pltpu.make_async_copy(ttab.at[tidx.at[b]], a_dst(b), asems[b]).wait()

  def grp(g, _):
    for b in range(NCH):
      pltpu.async_copy(ttab.at[tidx.at[g * NCH + b]], a_dst(b), asems[b],
                       add=True)
    return 0

  lax.fori_loop(1, NSP, grp, 0)

  def drn(g, _):
    for b in range(NCH):
      pltpu.make_async_copy(ttab.at[tidx.at[b]], a_dst(b), asems[b]).wait()
    return 0

  lax.fori_loop(1, NSP, drn, 0)

  pltpu.sync_copy(acc, t_out.at[pl.ds(base, RPW), :])


@jax.jit
def _sc_emb(uids, iids, spt, utab, itab, ttab):
  mesh = plsc.VectorSubcoreMesh(core_axis_name="c", subcore_axis_name="s")
  f = pl.kernel(
      _sc_emb_body,
      out_type=(
          jax.ShapeDtypeStruct((B, D), jnp.float32),
          jax.ShapeDtypeStruct((B, D), jnp.float32),
          jax.ShapeDtypeStruct((B, D), jnp.float32),
      ),
      mesh=mesh,
      compiler_params=pltpu.CompilerParams(needs_layout_passes=False),
      scratch_types=(
          pltpu.VMEM((NCH, CH), jnp.int32),
          pltpu.VMEM((NCH, CH), jnp.int32),
          pltpu.VMEM((TPW, CH), jnp.int32),
          pltpu.VMEM((CH, D), jnp.float32),
          pltpu.VMEM((CH, D), jnp.float32),
          pltpu.VMEM((CH, D), jnp.float32),
          pltpu.VMEM((CH, D), jnp.float32),
          pltpu.VMEM((RPW, D), jnp.float32),
      ) + (pltpu.SemaphoreType.DMA,) * 19,
  )
  return f(uids, iids, spt, utab, itab, ttab)


def _mlp_body(dense, u, i, t, bw1, bb1, bw2, bb2,
              t1u, t1i, t1t, t1d, tb1, tw2, tb2, out):
  f32 = jnp.float32
  h = jnp.maximum(jnp.dot(dense[...], bw1[...], preferred_element_type=f32)
                  + bb1[...], 0.0)
  d = jnp.dot(h, bw2[...], preferred_element_type=f32) + bb2[...]
  z = (jnp.dot(d, t1d[...], preferred_element_type=f32)
       + jnp.dot(u[...], t1u[...], preferred_element_type=f32)
       + jnp.dot(i[...], t1i[...], preferred_element_type=f32)
       + jnp.dot(t[...], t1t[...], preferred_element_type=f32)
       + tb1[...])
  z = jnp.maximum(z, 0.0)
  raw = jnp.dot(z, tw2[...], preferred_element_type=f32) + tb2[...]
  out[...] = jax.nn.sigmoid(raw)


BM = 2048


def _row_spec(w):
  return pl.BlockSpec((BM, w), lambda m: (m, 0))


def _full(a):
  return pl.BlockSpec(a.shape, lambda m: tuple(0 for _ in a.shape))


@jax.jit
def _mlp(dense, u, i, t, bw1, bb1, bw2, bb2,
         t1u, t1i, t1t, t1d, tb1, tw2, tb2):
  return pl.pallas_call(
      _mlp_body,
      grid=(B // BM,),
      in_specs=[
          _row_spec(DENSE), _row_spec(D), _row_spec(D), _row_spec(D),
          _full(bw1), _full(bb1), _full(bw2), _full(bb2),
          _full(t1u), _full(t1i), _full(t1t), _full(t1d), _full(tb1),
          _full(tw2), _full(tb2),
      ],
      out_specs=pl.BlockSpec((BM, 1), lambda m: (m, 0)),
      out_shape=jax.ShapeDtypeStruct((B, 1), jnp.float32),
  )(dense, u, i, t, bw1, bb1, bw2, bb2,
    t1u, t1i, t1t, t1d, tb1, tw2, tb2)


def kernel(user_ids, item_ids, dense_features, sparse_features,
           user_table, item_table, tag_table,
           bw1, bb1, bw2, bb2, tw1, tb1, tw2, tb2):
  uids = user_ids.astype(jnp.int32).reshape(B // CH, CH)
  iids = item_ids.astype(jnp.int32).reshape(B // CH, CH)
  spt = (sparse_features.astype(jnp.int32).T
         .reshape(NSP, NW, NCH, CH)
         .transpose(1, 0, 2, 3)
         .reshape(NW * TPW, CH))
  u_emb, i_emb, t_emb = _sc_emb(uids, iids, spt,
                                user_table, item_table, tag_table)
  t1u = tw1[0:D]
  t1i = tw1[D:2 * D]
  t1t = tw1[2 * D:3 * D]
  t1d = tw1[3 * D:4 * D]
  return _mlp(dense_features, u_emb, i_emb, t_emb,
              bw1, bb1.reshape(1, -1), bw2, bb2.reshape(1, -1),
              t1u, t1i, t1t, t1d, tb1.reshape(1, -1), tw2,
              tb2.reshape(1, -1))


# speedup vs baseline: 1.0095x; 1.0095x over previous
import jax
import jax.numpy as jnp
from jax import lax
from jax.experimental import pallas as pl
from jax.experimental.pallas import tpu as pltpu
from jax.experimental.pallas import tpu_sc as plsc

B = 16384
D = 128
NSP = 26
DENSE = 256
NC = 2
NS = 16
NW = NC * NS
RPW = B // NW
CH = 64
NCH = RPW // CH
TPW = NSP * NCH
NB = 4
NPLAIN = 3 * NCH


def _sc_emb_body(uids, iids, spt, utab, itab, ttab,
                 u_out, i_out, t_out,
                 idxu, idxi, tidx, gb0, gb1, gb2, gb3, acc,
                 tsem, isem0, isem1,
                 gsem0, gsem1, gsem2, gsem3,
                 ssem0, ssem1, ssem2, ssem3,
                 asem0, asem1, asem2, asem3,
                 asem4, asem5, asem6, asem7):
  c = lax.axis_index("c")
  s = lax.axis_index("s")
  wid = s * NC + c
  base = wid * RPW
  bufs = (gb0, gb1, gb2, gb3)
  gsems = (gsem0, gsem1, gsem2, gsem3)
  ssems = (ssem0, ssem1, ssem2, ssem3)
  asems = (asem0, asem1, asem2, asem3, asem4, asem5, asem6, asem7)

  pltpu.async_copy(spt.at[pl.ds(wid * TPW, TPW)], tidx, tsem)
  pltpu.async_copy(uids.at[pl.ds(wid * NCH, NCH)], idxu, isem0)
  pltpu.async_copy(iids.at[pl.ds(wid * NCH, NCH)], idxi, isem1)
  pltpu.make_async_copy(uids.at[pl.ds(wid * NCH, NCH)], idxu, isem0).wait()
  pltpu.make_async_copy(iids.at[pl.ds(wid * NCH, NCH)], idxi, isem1).wait()
  pltpu.make_async_copy(spt.at[pl.ds(wid * TPW, TPW)], tidx, tsem).wait()

  def a_dst(b):
    return acc.at[pl.ds(b * CH, CH), :]

  for b in range(NCH):
    pltpu.async_copy(ttab.at[tidx.at[b]], a_dst(b), asems[b])

  def g_src(k):
    if k < NCH:
      return utab.at[idxu.at[k]]
    return itab.at[idxi.at[k - NCH]]

  def o_dst(k):
    if k < NCH:
      return u_out.at[pl.ds(base + k * CH, CH), :]
    return i_out.at[pl.ds(base + (k - NCH) * CH, CH), :]

  NUI = 2 * NCH
  for b in range(NB):
    pltpu.async_copy(g_src(b), bufs[b], gsems[b])
  for k0 in range(0, NUI, NB):
    for b in range(NB):
      k = k0 + b
      pltpu.make_async_copy(g_src(k), bufs[b], gsems[b]).wait()
      pltpu.async_copy(bufs[b], o_dst(k), ssems[b])
    for b in range(NB):
      k = k0 + b
      pltpu.make_async_copy(bufs[b], o_dst(k), ssems[b]).wait()
      if k + NB < NUI:
        pltpu.async_copy(g_src(k + NB), bufs[b], gsems[b])

  for b in range(NCH):
    pltpu.make_async_copy(ttab.at[tidx.at[b]], a_dst(b), asems[b]).wait()

  def grp(g, _):
    for b in range(NCH):
      pltpu.async_copy(ttab.at[tidx.at[g * NCH + b]], a_dst(b), asems[b],
                       add=True)
    return 0

  lax.fori_loop(1, NSP, grp, 0)

  def drn(g, _):
    for b in range(NCH):
      pltpu.make_async_copy(ttab.at[tidx.at[b]], a_dst(b), asems[b]).wait()
    return 0

  lax.fori_loop(1, NSP, drn, 0)

  pltpu.sync_copy(acc, t_out.at[pl.ds(base, RPW), :])


@jax.jit
def _sc_emb(uids, iids, spt, utab, itab, ttab):
  mesh = plsc.VectorSubcoreMesh(core_axis_name="c", subcore_axis_name="s")
  f = pl.kernel(
      _sc_emb_body,
      out_type=(
          jax.ShapeDtypeStruct((B, D), jnp.float32),
          jax.ShapeDtypeStruct((B, D), jnp.float32),
          jax.ShapeDtypeStruct((B, D), jnp.float32),
      ),
      mesh=mesh,
      compiler_params=pltpu.CompilerParams(needs_layout_passes=False),
      scratch_types=(
          pltpu.VMEM((NCH, CH), jnp.int32),
          pltpu.VMEM((NCH, CH), jnp.int32),
          pltpu.VMEM((TPW, CH), jnp.int32),
          pltpu.VMEM((CH, D), jnp.float32),
          pltpu.VMEM((CH, D), jnp.float32),
          pltpu.VMEM((CH, D), jnp.float32),
          pltpu.VMEM((CH, D), jnp.float32),
          pltpu.VMEM((RPW, D), jnp.float32),
      ) + (pltpu.SemaphoreType.DMA,) * 19,
  )
  return f(uids, iids, spt, utab, itab, ttab)


def _mlp_body(dense, u, i, t, bw1, bb1, bw2, bb2,
              t1u, t1i, t1t, t1d, tb1, tw2, tb2, out):
  f32 = jnp.float32
  h = jnp.maximum(jnp.dot(dense[...], bw1[...], preferred_element_type=f32)
                  + bb1[...], 0.0)
  d = jnp.dot(h, bw2[...], preferred_element_type=f32) + bb2[...]
  z = (jnp.dot(d, t1d[...], preferred_element_type=f32)
       + jnp.dot(u[...], t1u[...], preferred_element_type=f32)
       + jnp.dot(i[...], t1i[...], preferred_element_type=f32)
       + jnp.dot(t[...], t1t[...], preferred_element_type=f32)
       + tb1[...])
  z = jnp.maximum(z, 0.0)
  raw = jnp.dot(z, tw2[...], preferred_element_type=f32) + tb2[...]
  out[...] = jax.nn.sigmoid(raw)


BM = 4096


def _row_spec(w):
  return pl.BlockSpec((BM, w), lambda m: (m, 0))


def _full(a):
  return pl.BlockSpec(a.shape, lambda m: tuple(0 for _ in a.shape))


@jax.jit
def _mlp(dense, u, i, t, bw1, bb1, bw2, bb2,
         t1u, t1i, t1t, t1d, tb1, tw2, tb2):
  return pl.pallas_call(
      _mlp_body,
      grid=(B // BM,),
      in_specs=[
          _row_spec(DENSE), _row_spec(D), _row_spec(D), _row_spec(D),
          _full(bw1), _full(bb1), _full(bw2), _full(bb2),
          _full(t1u), _full(t1i), _full(t1t), _full(t1d), _full(tb1),
          _full(tw2), _full(tb2),
      ],
      out_specs=pl.BlockSpec((BM, 1), lambda m: (m, 0)),
      out_shape=jax.ShapeDtypeStruct((B, 1), jnp.float32),
  )(dense, u, i, t, bw1, bb1, bw2, bb2,
    t1u, t1i, t1t, t1d, tb1, tw2, tb2)


def kernel(user_ids, item_ids, dense_features, sparse_features,
           user_table, item_table, tag_table,
           bw1, bb1, bw2, bb2, tw1, tb1, tw2, tb2):
  uids = user_ids.astype(jnp.int32).reshape(B // CH, CH)
  iids = item_ids.astype(jnp.int32).reshape(B // CH, CH)
  spt = (sparse_features.astype(jnp.int32).T
         .reshape(NSP, NW, NCH, CH)
         .transpose(1, 0, 2, 3)
         .reshape(NW * TPW, CH))
  u_emb, i_emb, t_emb = _sc_emb(uids, iids, spt,
                                user_table, item_table, tag_table)
  t1u = tw1[0:D]
  t1i = tw1[D:2 * D]
  t1t = tw1[2 * D:3 * D]
  t1d = tw1[3 * D:4 * D]
  return _mlp(dense_features, u_emb, i_emb, t_emb,
              bw1, bb1.reshape(1, -1), bw2, bb2.reshape(1, -1),
              t1u, t1i, t1t, t1d, tb1.reshape(1, -1), tw2,
              tb2.reshape(1, -1))
